# Initial kernel scaffold; baseline (speedup 1.0000x reference)
import functools

import jax
import jax.numpy as jnp
from jax import lax
from jax.experimental import pallas as pl
from jax.experimental.pallas import tpu as pltpu
from jax.experimental.pallas import tpu_sc as plsc

EMBED_DIM = 32
NC = 2
NS = 16
NW = NC * NS

ROW = 128
SUB = 8
CHUNK = ROW * SUB


@functools.partial(jax.jit, static_argnums=(2,))
def _gather(idx2d, table, total):
    n_chunks = total // (NW * CHUNK)
    mesh = plsc.VectorSubcoreMesh(core_axis_name="c", subcore_axis_name="s")

    @functools.partial(
        pl.kernel,
        mesh=mesh,
        out_type=jax.ShapeDtypeStruct((total, EMBED_DIM), jnp.float32),
        scratch_types=[
            pltpu.VMEM((SUB, ROW), jnp.int32),
            pltpu.VMEM((CHUNK, EMBED_DIM), jnp.float32),
            pltpu.SemaphoreType.DMA,
        ],
    )
    def body(idx_hbm, table_hbm, out_hbm, idx_v, rows_v, sem):
        wid = lax.axis_index("s") * NC + lax.axis_index("c")
        base_row = wid * (n_chunks * SUB)

        def step(g, carry):
            r0 = base_row + g * SUB
            pltpu.sync_copy(idx_hbm.at[pl.ds(r0, SUB)], idx_v)
            copies = [
                pltpu.async_copy(
                    table_hbm.at[idx_v.at[j]],
                    rows_v.at[pl.ds(j * ROW, ROW)],
                    sem,
                )
                for j in range(SUB)
            ]
            for c in copies:
                c.wait()
            pltpu.sync_copy(rows_v, out_hbm.at[pl.ds(r0 * ROW, CHUNK)])
            return carry

        lax.fori_loop(0, n_chunks, step, 0)

    return body(idx2d, table)


def kernel(input, suffixed, pref, chrs, table):
    B, L = input.shape
    total = B * L
    idx2d = input.reshape(total // ROW, ROW).astype(jnp.int32)
    out = _gather(idx2d, table, total)
    return out.reshape(B, L, EMBED_DIM)


# speedup vs baseline: 1.4576x; 1.4576x over previous
import functools

import jax
import jax.numpy as jnp
from jax import lax
from jax.experimental import pallas as pl
from jax.experimental.pallas import tpu as pltpu
from jax.experimental.pallas import tpu_sc as plsc

EMBED_DIM = 32
NC = 2
NS = 16
NW = NC * NS

ROW = 128
SUB = 8
CHUNK = ROW * SUB


@functools.partial(jax.jit, static_argnums=(2,))
def _gather(idx2d, table, total):
    n_chunks = total // (NW * CHUNK)
    mesh = plsc.VectorSubcoreMesh(core_axis_name="c", subcore_axis_name="s")

    @functools.partial(
        pl.kernel,
        mesh=mesh,
        out_type=jax.ShapeDtypeStruct((total, EMBED_DIM), jnp.float32),
        scratch_types=[
            pltpu.VMEM((SUB, ROW), jnp.int32),
            pltpu.VMEM((CHUNK, EMBED_DIM), jnp.float32),
            pltpu.SemaphoreType.DMA,
        ],
        compiler_params=pltpu.CompilerParams(use_tc_tiling_on_sc=False),
    )
    def body(idx_hbm, table_hbm, out_hbm, idx_v, rows_v, sem):
        wid = lax.axis_index("s") * NC + lax.axis_index("c")
        base_row = wid * (n_chunks * SUB)

        def step(g, carry):
            r0 = base_row + g * SUB
            pltpu.sync_copy(idx_hbm.at[pl.ds(r0, SUB)], idx_v)
            copies = [
                pltpu.async_copy(
                    table_hbm.at[idx_v.at[j]],
                    rows_v.at[pl.ds(j * ROW, ROW)],
                    sem,
                )
                for j in range(SUB)
            ]
            for c in copies:
                c.wait()
            pltpu.sync_copy(rows_v, out_hbm.at[pl.ds(r0 * ROW, CHUNK)])
            return carry

        lax.fori_loop(0, n_chunks, step, 0)

    return body(idx2d, table)


def kernel(input, suffixed, pref, chrs, table):
    B, L = input.shape
    total = B * L
    idx2d = input.reshape(total // ROW, ROW).astype(jnp.int32)
    out = _gather(idx2d, table, total)
    return out.reshape(B, L, EMBED_DIM)


# speedup vs baseline: 1.4981x; 1.0277x over previous
import functools

import jax
import jax.numpy as jnp
from jax import lax
from jax.experimental import pallas as pl
from jax.experimental.pallas import tpu as pltpu
from jax.experimental.pallas import tpu_sc as plsc

EMBED_DIM = 32
NC = 2
NS = 16
NW = NC * NS

ROW = 128
SUB = 8
CHUNK = ROW * SUB
NBUF = 3


@functools.partial(jax.jit, static_argnums=(2,))
def _gather(idx2d, table, total):
    per_tile = total // NW
    idx_rows = per_tile // ROW
    n_chunks = per_tile // CHUNK
    mesh = plsc.VectorSubcoreMesh(core_axis_name="c", subcore_axis_name="s")

    @functools.partial(
        pl.kernel,
        mesh=mesh,
        out_type=jax.ShapeDtypeStruct((total, EMBED_DIM), jnp.float32),
        scratch_types=[
            pltpu.VMEM((idx_rows, ROW), jnp.int32),
            [pltpu.VMEM((CHUNK, EMBED_DIM), jnp.float32) for _ in range(NBUF)],
            [pltpu.SemaphoreType.DMA for _ in range(NBUF)],
            [pltpu.SemaphoreType.DMA for _ in range(NBUF)],
        ],
        compiler_params=pltpu.CompilerParams(use_tc_tiling_on_sc=False),
    )
    def body(idx_hbm, table_hbm, out_hbm, idx_v, rows, gsem, wsem):
        wid = lax.axis_index("s") * NC + lax.axis_index("c")
        base_row = wid * idx_rows

        pltpu.sync_copy(idx_hbm.at[pl.ds(base_row, idx_rows)], idx_v)

        def fire(n, p):
            for j in range(SUB):
                pltpu.async_copy(
                    table_hbm.at[idx_v.at[n * SUB + j]],
                    rows[p].at[pl.ds(j * ROW, ROW)],
                    gsem[p],
                )

        def out_slice(n):
            return out_hbm.at[pl.ds((base_row + n * SUB) * ROW, CHUNK)]

        def drain_and_writeback(n, p):
            pltpu.make_async_copy(out_slice(n), rows[p], gsem[p]).wait()
            pltpu.async_copy(rows[p], out_slice(n), wsem[p])

        def wb_wait(n, p):
            pltpu.make_async_copy(rows[p], out_slice(n), wsem[p]).wait()

        n_slots = n_chunks + 1
        n_iters = (n_slots + NBUF - 1) // NBUF

        def step(it, carry):
            for b in range(NBUF):
                k = it * NBUF + b

                @pl.when(jnp.logical_and(k < n_chunks, k >= NBUF))
                def _():
                    wb_wait(k - NBUF, b)

                @pl.when(k < n_chunks)
                def _():
                    fire(k, b)

                @pl.when(jnp.logical_and(k >= 1, k < n_slots))
                def _():
                    drain_and_writeback(k - 1, (b - 1) % NBUF)

            return carry

        lax.fori_loop(0, n_iters, step, 0)

        for m in range(NBUF):
            n = n_chunks - 1 - m
            wb_wait(n, n % NBUF)

    return body(idx2d, table)


def kernel(input, suffixed, pref, chrs, table):
    B, L = input.shape
    total = B * L
    idx2d = input.reshape(total // ROW, ROW).astype(jnp.int32)
    out = _gather(idx2d, table, total)
    return out.reshape(B, L, EMBED_DIM)


# speedup vs baseline: 2.0489x; 1.3677x over previous
import functools

import jax
import jax.numpy as jnp
from jax import lax
from jax.experimental import pallas as pl
from jax.experimental.pallas import tpu as pltpu
from jax.experimental.pallas import tpu_sc as plsc

EMBED_DIM = 32
NC = 2
NS = 16
NW = NC * NS

ROW = 128
SUB = 8
CHUNK = ROW * SUB
NBUF = 3


@functools.partial(jax.jit, static_argnums=(2,))
def _gather(idx2d, table, total):
    per_tile = total // NW
    idx_rows = per_tile // ROW
    n_chunks = per_tile // CHUNK
    mesh = plsc.VectorSubcoreMesh(core_axis_name="c", subcore_axis_name="s")

    @functools.partial(
        pl.kernel,
        mesh=mesh,
        out_type=jax.ShapeDtypeStruct((total, 128), jnp.float32),
        scratch_types=[
            pltpu.VMEM((idx_rows, ROW), jnp.int32),
            [pltpu.VMEM((CHUNK, EMBED_DIM), jnp.float32) for _ in range(NBUF)],
            [pltpu.SemaphoreType.DMA for _ in range(NBUF)],
        ],
        compiler_params=pltpu.CompilerParams(use_tc_tiling_on_sc=False),
    )
    def body(idx_hbm, table_hbm, out_hbm, idx_v, rows, gsem):
        wid = lax.axis_index("s") * NC + lax.axis_index("c")
        base_row = wid * idx_rows

        pltpu.sync_copy(idx_hbm.at[pl.ds(base_row, idx_rows)], idx_v)

        def fire(n, p):
            for j in range(SUB):
                pltpu.async_copy(
                    table_hbm.at[idx_v.at[n * SUB + j]],
                    rows[p].at[pl.ds(j * ROW, ROW)],
                    gsem[p],
                )

        def out_slice(n):
            return out_hbm.at[
                pl.ds((base_row + n * SUB) * ROW, CHUNK), pl.ds(0, EMBED_DIM)
            ]

        def drain_and_writeback(n, p):
            pltpu.make_async_copy(out_slice(n), rows[p], gsem[p]).wait()
            pltpu.sync_copy(rows[p], out_slice(n))

        n_slots = n_chunks + 1
        n_iters = (n_slots + NBUF - 1) // NBUF

        def step(it, carry):
            for b in range(NBUF):
                k = it * NBUF + b

                @pl.when(k < n_chunks)
                def _():
                    fire(k, b)

                @pl.when(jnp.logical_and(k >= 1, k < n_slots))
                def _():
                    drain_and_writeback(k - 1, (b - 1) % NBUF)

            return carry

        lax.fori_loop(0, n_iters, step, 0)

    return body(idx2d, table)


def kernel(input, suffixed, pref, chrs, table):
    B, L = input.shape
    total = B * L
    idx2d = input.reshape(total // ROW, ROW).astype(jnp.int32)
    wide = _gather(idx2d, table, total)
    out = jax.lax.slice(wide, (0, 0), (total, EMBED_DIM))
    return out.reshape(B, L, EMBED_DIM)
